# baseline (speedup 1.0000x reference)
import functools

import jax
import jax.numpy as jnp
from jax import lax
from jax.experimental import pallas as pl
from jax.experimental.pallas import tpu as pltpu
from jax.experimental.pallas import tpu_sc as plsc

_TOPIC_K = 128
_BATCH = 8192
_ONE_HOP = 32
_TWO_HOP = 64
_DOC_ROWS = 1 + _TWO_HOP
_DSTRIDE = 72

_NC = 2
_NS = 16
_NW = _NC * _NS
_EPW = _BATCH // _NW
_L = 16
_NV = _TOPIC_K // _L
_NBUF = 5


def _row_sum(rows_ref, slot, start, nrows, unroll):
    def body(i, acc):
        r = start + i * unroll
        new = list(acc)
        for u in range(unroll):
            for k in range(_NV):
                new[k] = new[k] + rows_ref[slot, r + u, pl.ds(k * _L, _L)]
        return tuple(new)

    zero = tuple(jnp.zeros((_L,), jnp.float32) for _ in range(_NV))
    return lax.fori_loop(0, nrows // unroll, body, zero)


def _sc_body(doc_tab, word_tab, didx_hbm, widx_hbm, out_hbm,
             didx_v, widx_v, rows_d, rows_w, out_v,
             sem_idx, *sems):
    wid = lax.axis_index("s") * _NC + lax.axis_index("c")
    base = wid * _EPW

    pltpu.async_copy(
        didx_hbm.at[pl.ds(base * _DSTRIDE, _EPW * _DSTRIDE)], didx_v,
        sem_idx).wait()
    pltpu.async_copy(
        widx_hbm.at[pl.ds(base * _ONE_HOP, _EPW * _ONE_HOP)], widx_v,
        sem_idx).wait()

    sem_d = sems[:_NBUF]
    sem_w = sems[_NBUF:]

    def didx_at(e):
        return didx_v.at[pl.ds(pl.multiple_of(e * _DSTRIDE, 8), _DOC_ROWS)]

    def widx_at(e):
        return widx_v.at[pl.ds(pl.multiple_of(e * _ONE_HOP, 8), _ONE_HOP)]

    def issue(e, slot):
        pltpu.async_copy(doc_tab.at[didx_at(e)], rows_d.at[slot], sem_d[slot])
        pltpu.async_copy(word_tab.at[widx_at(e)], rows_w.at[slot], sem_w[slot])

    def wait(slot):
        pltpu.make_async_copy(
            doc_tab.at[didx_at(0)], rows_d.at[slot], sem_d[slot]).wait()
        pltpu.make_async_copy(
            word_tab.at[widx_at(0)], rows_w.at[slot], sem_w[slot]).wait()

    def compute(e, slot):
        th = _row_sum(rows_d, slot, 1, _TWO_HOP, 8)
        oh = _row_sum(rows_w, slot, 0, _ONE_HOP, 8)
        acc = [
            rows_d[slot, 0, pl.ds(k * _L, _L)]
            + th[k] * (1.0 / _TWO_HOP)
            + oh[k] * (1.0 / _ONE_HOP)
            for k in range(_NV)
        ]
        lanes = lax.iota(jnp.int32, _L)

        def shuffle(x, st):
            return x.at[lanes ^ st].get(mode="promise_in_bounds")

        m = acc[0]
        for k in range(1, _NV):
            m = jnp.maximum(m, acc[k])
        for st in (8, 4, 2, 1):
            m = jnp.maximum(m, shuffle(m, st))
        ex = [jnp.exp(a - m) for a in acc]
        s = ex[0]
        for k in range(1, _NV):
            s = s + ex[k]
        for st in (8, 4, 2, 1):
            s = s + shuffle(s, st)
        r = 1.0 / s
        for k in range(_NV):
            out_v[e, pl.ds(k * _L, _L)] = ex[k] * r

    for p in range(_NBUF - 1):
        issue(p, p)

    def group(i, carry):
        e0 = i * _NBUF
        for par in range(_NBUF):
            e = e0 + par

            @pl.when(e + _NBUF - 1 < _EPW)
            def _():
                issue(e + _NBUF - 1, (par + _NBUF - 1) % _NBUF)

            wait(par)
            compute(e, par)
        return carry

    lax.fori_loop(0, _EPW // _NBUF, group, 0)

    for e in range((_EPW // _NBUF) * _NBUF, _EPW):
        wait(e % _NBUF)
        compute(e, e % _NBUF)

    pltpu.async_copy(out_v, out_hbm.at[pl.ds(base, _EPW)], sem_idx).wait()


@functools.partial(
    pl.kernel,
    out_type=jax.ShapeDtypeStruct((_BATCH, _TOPIC_K), jnp.float32),
    mesh=plsc.VectorSubcoreMesh(core_axis_name="c", subcore_axis_name="s"),
    scratch_types=[
        pltpu.VMEM((_EPW * _DSTRIDE,), jnp.int32),
        pltpu.VMEM((_EPW * _ONE_HOP,), jnp.int32),
        pltpu.VMEM((_NBUF, _DOC_ROWS, _TOPIC_K), jnp.float32),
        pltpu.VMEM((_NBUF, _ONE_HOP, _TOPIC_K), jnp.float32),
        pltpu.VMEM((_EPW, _TOPIC_K), jnp.float32),
    ] + [pltpu.SemaphoreType.DMA] * (1 + 2 * _NBUF),
)
def _topic_sc_kernel(doc_tab, word_tab, didx_hbm, widx_hbm, out_hbm, *rest):
    _sc_body(doc_tab, word_tab, didx_hbm, widx_hbm, out_hbm, *rest)


def kernel(v, one_hop_list, two_hop_list, doc_topic_dist, word_topic_dist):
    didx = jnp.concatenate(
        [v.astype(jnp.int32)[:, None], two_hop_list.astype(jnp.int32)], axis=1)
    didx = jnp.pad(didx, ((0, 0), (0, _DSTRIDE - _DOC_ROWS)))
    widx = one_hop_list.astype(jnp.int32)
    return _topic_sc_kernel(
        doc_topic_dist, word_topic_dist, didx.reshape(-1), widx.reshape(-1))


# speedup vs baseline: 1.1989x; 1.1989x over previous
import functools

import jax
import jax.numpy as jnp
from jax import lax
from jax.experimental import pallas as pl
from jax.experimental.pallas import tpu as pltpu
from jax.experimental.pallas import tpu_sc as plsc

_TOPIC_K = 128
_BATCH = 8192
_ONE_HOP = 32
_TWO_HOP = 64
_DOC_ROWS = 1 + _TWO_HOP
_DSTRIDE = 72

_NC = 2
_NS = 16
_NW = _NC * _NS
_EPW = _BATCH // _NW
_L = 16
_NV = _TOPIC_K // _L
_NBUF = 5


def _row_sum(rows_ref, slot, start, nrows, unroll):
    def body(i, acc):
        r = start + i * unroll
        new = list(acc)
        for u in range(unroll):
            for k in range(_NV):
                new[k] = new[k] + rows_ref[slot, r + u, pl.ds(k * _L, _L)]
        return tuple(new)

    zero = tuple(jnp.zeros((_L,), jnp.float32) for _ in range(_NV))
    return lax.fori_loop(0, nrows // unroll, body, zero)


def _sc_body(doc_tab, word_tab, didx_hbm, widx_hbm, out_hbm,
             didx_v, widx_v, rows_d, rows_w, out_v,
             sem_idx, *sems):
    wid = lax.axis_index("s") * _NC + lax.axis_index("c")
    base = wid * _EPW

    pltpu.async_copy(
        didx_hbm.at[pl.ds(base * _DSTRIDE, _EPW * _DSTRIDE)], didx_v,
        sem_idx).wait()
    pltpu.async_copy(
        widx_hbm.at[pl.ds(base * _ONE_HOP, _EPW * _ONE_HOP)], widx_v,
        sem_idx).wait()

    sem_d = sems[:_NBUF]
    sem_w = sems[_NBUF:]

    def didx_at(e):
        return didx_v.at[pl.ds(pl.multiple_of(e * _DSTRIDE, 8), _DOC_ROWS)]

    def widx_at(e):
        return widx_v.at[pl.ds(pl.multiple_of(e * _ONE_HOP, 8), _ONE_HOP)]

    def issue(e, slot):
        pltpu.async_copy(doc_tab.at[didx_at(e)], rows_d.at[slot], sem_d[slot])
        pltpu.async_copy(word_tab.at[widx_at(e)], rows_w.at[slot], sem_w[slot])

    def wait(slot):
        pltpu.make_async_copy(
            doc_tab.at[didx_at(0)], rows_d.at[slot], sem_d[slot]).wait()
        pltpu.make_async_copy(
            word_tab.at[widx_at(0)], rows_w.at[slot], sem_w[slot]).wait()

    def compute(e, slot):
        th = _row_sum(rows_d, slot, 1, _TWO_HOP, 4)
        oh = _row_sum(rows_w, slot, 0, _ONE_HOP, 4)
        acc = [
            rows_d[slot, 0, pl.ds(k * _L, _L)]
            + th[k] * (1.0 / _TWO_HOP)
            + oh[k] * (1.0 / _ONE_HOP)
            for k in range(_NV)
        ]
        lanes = lax.iota(jnp.int32, _L)

        def shuffle(x, st):
            return x.at[lanes ^ st].get(mode="promise_in_bounds")

        m = acc[0]
        for k in range(1, _NV):
            m = jnp.maximum(m, acc[k])
        for st in (8, 4, 2, 1):
            m = jnp.maximum(m, shuffle(m, st))
        ex = [jnp.exp(a - m) for a in acc]
        s = ex[0]
        for k in range(1, _NV):
            s = s + ex[k]
        for st in (8, 4, 2, 1):
            s = s + shuffle(s, st)
        r = 1.0 / s
        for k in range(_NV):
            out_v[e, pl.ds(k * _L, _L)] = ex[k] * r

    for p in range(_NBUF - 1):
        issue(p, p)

    def group(i, carry):
        e0 = i * _NBUF
        for par in range(_NBUF):
            e = e0 + par

            @pl.when(e + _NBUF - 1 < _EPW)
            def _():
                issue(e + _NBUF - 1, (par + _NBUF - 1) % _NBUF)

            wait(par)
            compute(e, par)
        return carry

    lax.fori_loop(0, _EPW // _NBUF, group, 0)

    for e in range((_EPW // _NBUF) * _NBUF, _EPW):
        wait(e % _NBUF)
        compute(e, e % _NBUF)

    pltpu.async_copy(out_v, out_hbm.at[pl.ds(base, _EPW)], sem_idx).wait()


@functools.partial(
    pl.kernel,
    out_type=jax.ShapeDtypeStruct((_BATCH, _TOPIC_K), jnp.float32),
    mesh=plsc.VectorSubcoreMesh(core_axis_name="c", subcore_axis_name="s"),
    scratch_types=[
        pltpu.VMEM((_EPW * _DSTRIDE,), jnp.int32),
        pltpu.VMEM((_EPW * _ONE_HOP,), jnp.int32),
        pltpu.VMEM((_NBUF, _DOC_ROWS, _TOPIC_K), jnp.float32),
        pltpu.VMEM((_NBUF, _ONE_HOP, _TOPIC_K), jnp.float32),
        pltpu.VMEM((_EPW, _TOPIC_K), jnp.float32),
    ] + [pltpu.SemaphoreType.DMA] * (1 + 2 * _NBUF),
)
def _topic_sc_kernel(doc_tab, word_tab, didx_hbm, widx_hbm, out_hbm, *rest):
    _sc_body(doc_tab, word_tab, didx_hbm, widx_hbm, out_hbm, *rest)


def kernel(v, one_hop_list, two_hop_list, doc_topic_dist, word_topic_dist):
    didx = jnp.concatenate(
        [v.astype(jnp.int32)[:, None], two_hop_list.astype(jnp.int32)], axis=1)
    didx = jnp.pad(didx, ((0, 0), (0, _DSTRIDE - _DOC_ROWS)))
    widx = one_hop_list.astype(jnp.int32)
    return _topic_sc_kernel(
        doc_topic_dist, word_topic_dist, didx.reshape(-1), widx.reshape(-1))
